# baseline (speedup 1.0000x reference)
import functools

import jax
import jax.numpy as jnp
from jax import lax
from jax.experimental import pallas as pl
from jax.experimental.pallas import tpu as pltpu
from jax.experimental.pallas import tpu_sc as plsc

_DIM = 64
_NC, _NS = 2, 16
_CHUNK = 128
_H = 524288


def _skipgram_body(V, B, K, bpt,
                   ttT, ctT, tgt_hbm, ctx_hbm, out_hbm,
                   buf_loA, buf_loB, buf_hi, orig_t, orig_c,
                   tidx_v, tpos_v, cidx_v, cpos_v,
                   tvals_v, cvals_v, acc_v,
                   sem0, sem1, sem2, sem3, sem_g):
    core = lax.axis_index("c")
    sid = lax.axis_index("s")
    dpc = _DIM // _NC
    d0 = core * dpc
    b0 = sid * bpt
    sems = [sem0, sem1, sem2, sem3]
    nct = bpt
    ncc = K * bpt

    pltpu.sync_copy(tgt_hbm.at[pl.ds(b0, bpt)], orig_t)
    for k in range(K):
        pltpu.sync_copy(ctx_hbm.at[pl.ds(k * B + b0, bpt)],
                        orig_c.at[pl.ds(k * bpt, bpt)])
    glist_v = orig_c

    def prefill(buf, size, val):
        def b_(i, _):
            buf[pl.ds(i * 16, 16)] = jnp.full((16,), val, jnp.int32)
            return 0
        lax.fori_loop(0, size // 16, b_, 0)

    prefill(tidx_v, tidx_v.shape[0], 0)
    prefill(cidx_v, cidx_v.shape[0], 0)
    prefill(tpos_v, tpos_v.shape[0], nct)
    prefill(cpos_v, cpos_v.shape[0], ncc)

    def bin_list(orig, n, idxb, posb, cast=False):
        def load(i):
            v = orig[pl.ds(i * 16, 16)]
            return plsc.bitcast(v, jnp.int32) if cast else v

        def count_body(i, acc):
            v = load(i)
            return acc + jnp.sum((v < _H).astype(jnp.int32))
        nlo = lax.fori_loop(0, n // 16, count_body, 0)
        hi0 = ((nlo + _CHUNK - 1) // _CHUNK) * _CHUNK

        def fill_body(i, cur):
            clo, chi = cur
            v = load(i)
            m = v < _H
            nm = jnp.logical_not(m)
            pos = i * 16 + lax.iota(jnp.int32, 16)
            plsc.store_compressed(idxb.at[pl.ds(clo, 16)], v, mask=m)
            plsc.store_compressed(posb.at[pl.ds(clo, 16)], pos, mask=m)
            plsc.store_compressed(idxb.at[pl.ds(chi, 16)], v - _H, mask=nm)
            plsc.store_compressed(posb.at[pl.ds(chi, 16)], pos, mask=nm)
            nl = jnp.sum(m.astype(jnp.int32))
            return (clo + nl, chi + 16 - nl)

        lax.fori_loop(0, n // 16, fill_body, (0, hi0))
        return nlo, hi0

    nlo_t, hi0_t = bin_list(orig_t, nct, tidx_v, tpos_v)
    nlo_c, hi0_c = bin_list(orig_c, ncc, cidx_v, cpos_v, cast=True)

    def zero_body(i, _):
        acc_v[pl.ds(i * 16, 16)] = jnp.zeros((16,), jnp.float32)
        return 0

    lax.fori_loop(0, ncc // 16, zero_body, 0)

    _DSTS = {0: "loA", 1: "hi", 2: "loB", 3: "hi"}

    def _refs(k):
        table = ttT if k < 2 else ctT
        dst = {"loA": buf_loA, "loB": buf_loB, "hi": buf_hi}[_DSTS[k]]
        if k % 2 == 0:
            src = lambda d: table.at[d].at[pl.ds(0, _H)]
        else:
            src = lambda d: table.at[d].at[pl.ds(_H, V - _H)]
        return src, dst

    def stage(k, d):
        src, dst = _refs(k)
        pltpu.async_copy(src(d), dst, sems[k])

    def stage_wait(k, d):
        src, dst = _refs(k)
        pltpu.make_async_copy(src(d), dst, sems[k]).wait()

    def gather_piece(buf, idxb, posb, start, cnt, vals):
        nch = (cnt + _CHUNK - 1) // _CHUNK

        def fire(j, _):
            pltpu.async_copy(
                buf.at[idxb.at[pl.ds(start + j * _CHUNK, _CHUNK)]],
                glist_v.at[pl.ds(j * _CHUNK, _CHUNK)], sem_g)
            return 0

        def drain(j, _):
            pltpu.make_async_copy(
                buf.at[idxb.at[pl.ds(start + j * _CHUNK, _CHUNK)]],
                glist_v.at[pl.ds(j * _CHUNK, _CHUNK)], sem_g).wait()
            return 0

        lax.fori_loop(0, nch, fire, 0)
        lax.fori_loop(0, nch, drain, 0)

        def scat(i, _):
            val = glist_v[pl.ds(i * 16, 16)]
            p = posb[pl.ds(start + i * 16, 16)]
            m = (i * 16 + lax.iota(jnp.int32, 16)) < cnt
            plsc.store_scatter(vals, [p], val, mask=m)
            return 0

        lax.fori_loop(0, (cnt + 15) // 16, scat, 0)

    @pl.when(sid == 0)
    def _():
        stage(0, d0)

    @pl.when(sid == 1)
    def _():
        stage(1, d0)

    def i_body(i, _):
        sl = pl.ds(i * 16, 16)
        t = tvals_v[sl]
        for k in range(K):
            ksl = pl.ds(k * bpt + i * 16, 16)
            plsc.addupdate(acc_v.at[ksl], t * cvals_v[ksl])
        return 0

    def d_body(dd, _):
        d = d0 + dd

        @pl.when(sid == 0)
        def _():
            stage_wait(0, d)
        plsc.subcore_barrier()

        @pl.when(sid == 2)
        def _():
            stage(2, d)
        gather_piece(buf_loA, tidx_v, tpos_v, 0, nlo_t, tvals_v)

        @pl.when(sid == 1)
        def _():
            stage_wait(1, d)
        plsc.subcore_barrier()
        gather_piece(buf_hi, tidx_v, tpos_v, hi0_t, nct - nlo_t, tvals_v)
        plsc.subcore_barrier()

        @pl.when(sid == 3)
        def _():
            stage(3, d)

        @pl.when(sid == 2)
        def _():
            stage_wait(2, d)
        plsc.subcore_barrier()

        @pl.when(jnp.logical_and(sid == 0, dd + 1 < dpc))
        def _():
            stage(0, d + 1)
        gather_piece(buf_loB, cidx_v, cpos_v, 0, nlo_c, cvals_v)

        @pl.when(sid == 3)
        def _():
            stage_wait(3, d)
        plsc.subcore_barrier()
        gather_piece(buf_hi, cidx_v, cpos_v, hi0_c, ncc - nlo_c, cvals_v)
        plsc.subcore_barrier()

        @pl.when(jnp.logical_and(sid == 1, dd + 1 < dpc))
        def _():
            stage(1, d + 1)

        lax.fori_loop(0, bpt // 16, i_body, 0)
        return 0

    lax.fori_loop(0, dpc, d_body, 0)
    plsc.subcore_barrier()

    for k in range(K):
        pltpu.sync_copy(acc_v.at[pl.ds(k * bpt, bpt)],
                        out_hbm.at[pl.ds((core * K + k) * B + b0, bpt)])


def kernel(target, context, target_table, context_table):
    V, D = target_table.shape
    B = target.shape[0]
    K = context.shape[1]
    bpt = B // _NS
    ttT = target_table.T
    ctT = context_table.T
    tgt = target.reshape(B)
    ctx = jnp.transpose(context.reshape(B, K)).reshape(K * B)
    ctx = lax.bitcast_convert_type(ctx, jnp.float32)

    mesh = plsc.VectorSubcoreMesh(core_axis_name="c", subcore_axis_name="s",
                                  num_cores=_NC, num_subcores=_NS)
    tsz = bpt + 2 * _CHUNK
    csz = K * bpt + 2 * _CHUNK
    parts = pl.kernel(
        functools.partial(_skipgram_body, V, B, K, bpt),
        out_type=jax.ShapeDtypeStruct((_NC * K * B,), jnp.float32),
        mesh=mesh,
        compiler_params=pltpu.CompilerParams(needs_layout_passes=False),
        scratch_types=[
            pltpu.VMEM_SHARED((_H,), jnp.float32),
            pltpu.VMEM_SHARED((_H,), jnp.float32),
            pltpu.VMEM_SHARED((V - _H,), jnp.float32),
            pltpu.VMEM((bpt,), jnp.int32),
            pltpu.VMEM((K * bpt,), jnp.float32),
            pltpu.VMEM((tsz,), jnp.int32),
            pltpu.VMEM((tsz,), jnp.int32),
            pltpu.VMEM((csz,), jnp.int32),
            pltpu.VMEM((csz,), jnp.int32),
            pltpu.VMEM((bpt + 16,), jnp.float32),
            pltpu.VMEM((K * bpt + 16,), jnp.float32),
            pltpu.VMEM((K * bpt,), jnp.float32),
            pltpu.SemaphoreType.DMA,
            pltpu.SemaphoreType.DMA,
            pltpu.SemaphoreType.DMA,
            pltpu.SemaphoreType.DMA,
            pltpu.SemaphoreType.DMA,
        ],
    )(ttT, ctT, tgt, ctx)
    parts = parts.reshape(_NC, K, B)
    return jnp.transpose(parts[0] + parts[1])


# speedup vs baseline: 1.0232x; 1.0232x over previous
import functools

import jax
import jax.numpy as jnp
from jax import lax
from jax.experimental import pallas as pl
from jax.experimental.pallas import tpu as pltpu
from jax.experimental.pallas import tpu_sc as plsc

_DIM = 64
_NC, _NS = 2, 16
_CHUNK = 128
_H = 576000


def _skipgram_body(V, B, K, bpt,
                   ttT, ctT, tgt_hbm, ctx_hbm, out_hbm,
                   buf_loA, buf_loB, buf_hi, orig_t, orig_c,
                   tidx_v, tpos_v, cidx_v, cpos_v,
                   tvals_v, cvals_v, acc_v,
                   sem0, sem1, sem2, sem3, sem_g):
    core = lax.axis_index("c")
    sid = lax.axis_index("s")
    dpc = _DIM // _NC
    d0 = core * dpc
    b0 = sid * bpt
    sems = [sem0, sem1, sem2, sem3]
    nct = bpt
    ncc = K * bpt

    pltpu.sync_copy(tgt_hbm.at[pl.ds(b0, bpt)], orig_t)
    for k in range(K):
        pltpu.sync_copy(ctx_hbm.at[pl.ds(k * B + b0, bpt)],
                        orig_c.at[pl.ds(k * bpt, bpt)])
    glist_v = orig_c

    def prefill(buf, size, val):
        def b_(i, _):
            buf[pl.ds(i * 16, 16)] = jnp.full((16,), val, jnp.int32)
            return 0
        lax.fori_loop(0, size // 16, b_, 0)

    prefill(tidx_v, tidx_v.shape[0], 0)
    prefill(cidx_v, cidx_v.shape[0], 0)
    prefill(tpos_v, tpos_v.shape[0], nct)
    prefill(cpos_v, cpos_v.shape[0], ncc)

    def bin_list(orig, n, idxb, posb, cast=False):
        def load(i):
            v = orig[pl.ds(i * 16, 16)]
            return plsc.bitcast(v, jnp.int32) if cast else v

        def count_body(i, acc):
            v = load(i)
            return acc + jnp.sum((v < _H).astype(jnp.int32))
        nlo = lax.fori_loop(0, n // 16, count_body, 0)
        hi0 = ((nlo + _CHUNK - 1) // _CHUNK) * _CHUNK

        def fill_body(i, cur):
            clo, chi = cur
            v = load(i)
            m = v < _H
            nm = jnp.logical_not(m)
            pos = i * 16 + lax.iota(jnp.int32, 16)
            plsc.store_compressed(idxb.at[pl.ds(clo, 16)], v, mask=m)
            plsc.store_compressed(posb.at[pl.ds(clo, 16)], pos, mask=m)
            plsc.store_compressed(idxb.at[pl.ds(chi, 16)], v - _H, mask=nm)
            plsc.store_compressed(posb.at[pl.ds(chi, 16)], pos, mask=nm)
            nl = jnp.sum(m.astype(jnp.int32))
            return (clo + nl, chi + 16 - nl)

        lax.fori_loop(0, n // 16, fill_body, (0, hi0))
        return nlo, hi0

    nlo_t, hi0_t = bin_list(orig_t, nct, tidx_v, tpos_v)
    nlo_c, hi0_c = bin_list(orig_c, ncc, cidx_v, cpos_v, cast=True)

    def zero_body(i, _):
        acc_v[pl.ds(i * 16, 16)] = jnp.zeros((16,), jnp.float32)
        return 0

    lax.fori_loop(0, ncc // 16, zero_body, 0)

    _DSTS = {0: "loA", 1: "hi", 2: "loB", 3: "hi"}

    def _refs(k):
        table = ttT if k < 2 else ctT
        dst = {"loA": buf_loA, "loB": buf_loB, "hi": buf_hi}[_DSTS[k]]
        if k % 2 == 0:
            src = lambda d: table.at[d].at[pl.ds(0, _H)]
        else:
            src = lambda d: table.at[d].at[pl.ds(_H, V - _H)]
        return src, dst

    def stage(k, d):
        src, dst = _refs(k)
        pltpu.async_copy(src(d), dst, sems[k])

    def stage_wait(k, d):
        src, dst = _refs(k)
        pltpu.make_async_copy(src(d), dst, sems[k]).wait()

    def gather_piece(buf, idxb, posb, start, cnt, vals):
        nch = (cnt + _CHUNK - 1) // _CHUNK

        def fire(j, _):
            pltpu.async_copy(
                buf.at[idxb.at[pl.ds(start + j * _CHUNK, _CHUNK)]],
                glist_v.at[pl.ds(j * _CHUNK, _CHUNK)], sem_g)
            return 0

        def drain(j, _):
            pltpu.make_async_copy(
                buf.at[idxb.at[pl.ds(start + j * _CHUNK, _CHUNK)]],
                glist_v.at[pl.ds(j * _CHUNK, _CHUNK)], sem_g).wait()
            return 0

        lax.fori_loop(0, nch, fire, 0)
        lax.fori_loop(0, nch, drain, 0)

        def scat(i, _):
            val = glist_v[pl.ds(i * 16, 16)]
            p = posb[pl.ds(start + i * 16, 16)]
            m = (i * 16 + lax.iota(jnp.int32, 16)) < cnt
            plsc.store_scatter(vals, [p], val, mask=m)
            return 0

        lax.fori_loop(0, (cnt + 15) // 16, scat, 0)

    @pl.when(sid == 0)
    def _():
        stage(0, d0)

    @pl.when(sid == 1)
    def _():
        stage(1, d0)

    def i_body(i, _):
        sl = pl.ds(i * 16, 16)
        t = tvals_v[sl]
        for k in range(K):
            ksl = pl.ds(k * bpt + i * 16, 16)
            plsc.addupdate(acc_v.at[ksl], t * cvals_v[ksl])
        return 0

    def d_body(dd, _):
        d = d0 + dd

        @pl.when(sid == 0)
        def _():
            stage_wait(0, d)
        plsc.subcore_barrier()

        @pl.when(sid == 2)
        def _():
            stage(2, d)
        gather_piece(buf_loA, tidx_v, tpos_v, 0, nlo_t, tvals_v)

        @pl.when(sid == 1)
        def _():
            stage_wait(1, d)
        plsc.subcore_barrier()
        gather_piece(buf_hi, tidx_v, tpos_v, hi0_t, nct - nlo_t, tvals_v)
        plsc.subcore_barrier()

        @pl.when(sid == 3)
        def _():
            stage(3, d)

        @pl.when(sid == 2)
        def _():
            stage_wait(2, d)
        plsc.subcore_barrier()

        @pl.when(jnp.logical_and(sid == 0, dd + 1 < dpc))
        def _():
            stage(0, d + 1)
        gather_piece(buf_loB, cidx_v, cpos_v, 0, nlo_c, cvals_v)

        @pl.when(sid == 3)
        def _():
            stage_wait(3, d)
        plsc.subcore_barrier()
        gather_piece(buf_hi, cidx_v, cpos_v, hi0_c, ncc - nlo_c, cvals_v)
        plsc.subcore_barrier()

        @pl.when(jnp.logical_and(sid == 1, dd + 1 < dpc))
        def _():
            stage(1, d + 1)

        lax.fori_loop(0, bpt // 16, i_body, 0)
        return 0

    lax.fori_loop(0, dpc, d_body, 0)
    plsc.subcore_barrier()

    for k in range(K):
        pltpu.sync_copy(acc_v.at[pl.ds(k * bpt, bpt)],
                        out_hbm.at[pl.ds((core * K + k) * B + b0, bpt)])


def kernel(target, context, target_table, context_table):
    V, D = target_table.shape
    B = target.shape[0]
    K = context.shape[1]
    bpt = B // _NS
    ttT = target_table.T
    ctT = context_table.T
    tgt = target.reshape(B)
    ctx = jnp.transpose(context.reshape(B, K)).reshape(K * B)
    ctx = lax.bitcast_convert_type(ctx, jnp.float32)

    mesh = plsc.VectorSubcoreMesh(core_axis_name="c", subcore_axis_name="s",
                                  num_cores=_NC, num_subcores=_NS)
    tsz = bpt + 2 * _CHUNK
    csz = K * bpt + 2 * _CHUNK
    parts = pl.kernel(
        functools.partial(_skipgram_body, V, B, K, bpt),
        out_type=jax.ShapeDtypeStruct((_NC * K * B,), jnp.float32),
        mesh=mesh,
        compiler_params=pltpu.CompilerParams(needs_layout_passes=False),
        scratch_types=[
            pltpu.VMEM_SHARED((_H,), jnp.float32),
            pltpu.VMEM_SHARED((_H,), jnp.float32),
            pltpu.VMEM_SHARED((V - _H,), jnp.float32),
            pltpu.VMEM((bpt,), jnp.int32),
            pltpu.VMEM((K * bpt,), jnp.float32),
            pltpu.VMEM((tsz,), jnp.int32),
            pltpu.VMEM((tsz,), jnp.int32),
            pltpu.VMEM((csz,), jnp.int32),
            pltpu.VMEM((csz,), jnp.int32),
            pltpu.VMEM((bpt + 16,), jnp.float32),
            pltpu.VMEM((K * bpt + 16,), jnp.float32),
            pltpu.VMEM((K * bpt,), jnp.float32),
            pltpu.SemaphoreType.DMA,
            pltpu.SemaphoreType.DMA,
            pltpu.SemaphoreType.DMA,
            pltpu.SemaphoreType.DMA,
            pltpu.SemaphoreType.DMA,
        ],
    )(ttT, ctT, tgt, ctx)
    parts = parts.reshape(_NC, K, B)
    return jnp.transpose(parts[0] + parts[1])
